# baseline (speedup 1.0000x reference)
import functools
import math

import jax
import jax.numpy as jnp
from jax import lax
from jax.experimental import pallas as pl
from jax.experimental.pallas import tpu as pltpu


def _body(x_ref, bnd_ref, exp_ref, out_ref, xs_ref, xs_s_ref, *, n_rows, n, c,
          k, n_iters, group):
    j = pl.program_id(1)
    inv_sqrt_c = 1.0 / math.sqrt(c)
    _m = math.isqrt(c)
    exact_scale = _m * _m == c and _m > 0 and (_m & (_m - 1)) == 0

    @pl.when(j == 0)
    def _():
        xs_ref[...] = jnp.sum(x_ref[0], axis=0)
        xs_s_ref[...] = (xs_ref[...] * inv_sqrt_c if exact_scale
                         else xs_ref[...])

    lhs = xs_ref[:, pl.ds(j * n_rows, n_rows)]
    a = lax.dot_general(lhs, xs_s_ref[...], (((0,), (0,)), ((), ())),
                        preferred_element_type=jnp.float32)
    if not exact_scale:
        a = a / math.sqrt(c)
    r = jnp.maximum(a, 0.0)
    e1 = jnp.exp(r - jnp.max(r, axis=-1, keepdims=True))
    s = e1 * (1.0 / jnp.sum(e1, axis=-1, keepdims=True))
    e2 = jnp.exp(s - jnp.max(s, axis=-1, keepdims=True))
    w = e2 * (1.0 / jnp.sum(e2, axis=-1, keepdims=True))

    bits = lax.bitcast_convert_type(w, jnp.int32)

    def prefix_count(eq):
        zb = eq.astype(jnp.bfloat16)
        cnt = lax.dot_general(zb, bnd_ref[...], (((1,), (0,)), ((), ())),
                              preferred_element_type=jnp.float32)
        base = lax.dot_general(cnt, exp_ref[...], (((1,), (0,)), ((), ())),
                               preferred_element_type=jnp.float32,
                               precision=lax.Precision.HIGHEST)
        lane = lax.broadcasted_iota(jnp.int32, (1, n), 1) % group
        lp = zb
        d = 1
        while d < group:
            sh = lax.concatenate(
                [jnp.zeros((n_rows, d), jnp.bfloat16), lp[:, : n - d]], 1)
            lp = lp + sh * (lane >= d).astype(jnp.bfloat16)
            d *= 2
        return base + (lp - zb).astype(jnp.float32)

    w_min = jnp.min(w, axis=-1, keepdims=True)
    t0 = lax.bitcast_convert_type(w_min, jnp.int32)
    eq0 = bits == t0
    pc0 = prefix_count(eq0)
    e0 = pc0[:, n - 1 : n] + jnp.where(eq0[:, n - 1 : n], 1.0, 0.0)
    keep0 = (~eq0) | (pc0 < (k - n) + e0)
    out_ref[0] = jnp.where(keep0, w, 0.0)

    @pl.when(jnp.any(e0 <= n - k))
    def _():
        def search(i, carry):
            lo, hi = carry
            mid = (lo + hi) >> 1
            cnt = jnp.sum((bits >= mid).astype(jnp.int32), axis=-1,
                          keepdims=True)
            ge = cnt >= k
            return jnp.where(ge, mid, lo), jnp.where(ge, hi, mid)

        lo0 = jnp.zeros((n_rows, 1), jnp.int32)
        hi0 = jnp.full((n_rows, 1), 0x3F800001, jnp.int32)
        t = lax.fori_loop(0, n_iters, search, (lo0, hi0))[0]
        g = jnp.sum((bits > t).astype(jnp.int32), axis=-1, keepdims=True)
        eq = bits == t
        pc = prefix_count(eq)
        keep = (bits > t) | (eq & (pc < (k - g).astype(jnp.float32)))
        out_ref[0] = jnp.where(keep, w, 0.0)


def kernel(x):
    b, c, n, t = x.shape
    k = int(n * 0.8)
    n_rows = 512 if n % 512 == 0 else n
    xt = jnp.transpose(x, (0, 3, 1, 2))
    group = 16 if n % 16 == 0 else 1
    ngrp = n // group
    bnd = (jnp.arange(n)[:, None] < group * jnp.arange(ngrp)[None, :]
           ).astype(jnp.bfloat16)
    expand = (jnp.arange(ngrp)[:, None] == jnp.arange(n)[None, :] // group
              ).astype(jnp.float32)
    body = functools.partial(_body, n_rows=n_rows, n=n, c=c, k=k, n_iters=30,
                             group=group)
    return pl.pallas_call(
        body,
        grid=(b, n // n_rows),
        in_specs=[pl.BlockSpec((1, t, c, n), lambda bi, ji: (bi, 0, 0, 0)),
                  pl.BlockSpec((n, ngrp), lambda bi, ji: (0, 0)),
                  pl.BlockSpec((ngrp, n), lambda bi, ji: (0, 0))],
        out_specs=pl.BlockSpec((1, n_rows, n), lambda bi, ji: (bi, ji, 0)),
        out_shape=jax.ShapeDtypeStruct((b, n, n), jnp.float32),
        scratch_shapes=[pltpu.VMEM((c, n), jnp.float32),
                        pltpu.VMEM((c, n), jnp.float32)],
    )(xt, bnd, expand)


# speedup vs baseline: 2.0799x; 2.0799x over previous
import functools
import math

import jax
import jax.numpy as jnp
from jax import lax
from jax.experimental import pallas as pl
from jax.experimental.pallas import tpu as pltpu


def _body(x_ref, sut_ref, out_ref, xs_ref, xs_s_ref, *, n_rows, n, c,
          k, n_iters, chunk):
    j = pl.program_id(1)
    inv_sqrt_c = 1.0 / math.sqrt(c)
    _m = math.isqrt(c)
    exact_scale = _m * _m == c and _m > 0 and (_m & (_m - 1)) == 0

    @pl.when(j == 0)
    def _():
        xs_ref[...] = jnp.sum(x_ref[0], axis=0)
        xs_s_ref[...] = (xs_ref[...] * inv_sqrt_c if exact_scale
                         else xs_ref[...])

    lhs = xs_ref[:, pl.ds(j * n_rows, n_rows)]
    a = lax.dot_general(lhs, xs_s_ref[...], (((0,), (0,)), ((), ())),
                        preferred_element_type=jnp.float32)
    if not exact_scale:
        a = a / math.sqrt(c)
    r = jnp.maximum(a, 0.0)
    e1 = jnp.exp(r - jnp.max(r, axis=-1, keepdims=True))
    s = e1 * (1.0 / jnp.sum(e1, axis=-1, keepdims=True))
    e2 = jnp.exp(s - jnp.max(s, axis=-1, keepdims=True))
    w = e2 * (1.0 / jnp.sum(e2, axis=-1, keepdims=True))

    bits = lax.bitcast_convert_type(w, jnp.int32)

    def prefix_count(eq):
        zb = eq.astype(jnp.bfloat16)
        base = jnp.zeros((n_rows, 1), jnp.float32)
        pieces = []
        for q in range(n // chunk):
            zc = zb[:, q * chunk : (q + 1) * chunk]
            p = lax.dot_general(zc, sut_ref[...], (((1,), (0,)), ((), ())),
                                preferred_element_type=jnp.float32)
            pieces.append(p + base)
            base = base + (p[:, chunk - 1 : chunk]
                           + zc[:, chunk - 1 : chunk].astype(jnp.float32))
        pc = pieces[0] if len(pieces) == 1 else lax.concatenate(pieces, 1)
        return pc, base

    w_min = jnp.min(w, axis=-1, keepdims=True)
    t0 = lax.bitcast_convert_type(w_min, jnp.int32)
    eq0 = bits == t0
    pc0, e0 = prefix_count(eq0)
    keep0 = (~eq0) | (pc0 < (k - n) + e0)
    out_ref[0] = jnp.where(keep0, w, 0.0)

    @pl.when(jnp.any(e0 <= n - k))
    def _():
        def search(i, carry):
            lo, hi = carry
            mid = (lo + hi) >> 1
            cnt = jnp.sum((bits >= mid).astype(jnp.int32), axis=-1,
                          keepdims=True)
            ge = cnt >= k
            return jnp.where(ge, mid, lo), jnp.where(ge, hi, mid)

        lo0 = jnp.zeros((n_rows, 1), jnp.int32)
        hi0 = jnp.full((n_rows, 1), 0x3F800001, jnp.int32)
        t = lax.fori_loop(0, n_iters, search, (lo0, hi0))[0]
        g = jnp.sum((bits > t).astype(jnp.int32), axis=-1, keepdims=True)
        eq = bits == t
        pc, _ = prefix_count(eq)
        keep = (bits > t) | (eq & (pc < (k - g).astype(jnp.float32)))
        out_ref[0] = jnp.where(keep, w, 0.0)


def kernel(x):
    b, c, n, t = x.shape
    k = int(n * 0.8)
    n_rows = 1024 if n % 1024 == 0 else n
    xt = jnp.transpose(x, (0, 3, 1, 2))
    chunk = 256 if n % 256 == 0 else n
    sut = (jnp.arange(chunk)[:, None] < jnp.arange(chunk)[None, :]
           ).astype(jnp.bfloat16)
    body = functools.partial(_body, n_rows=n_rows, n=n, c=c, k=k, n_iters=30,
                             chunk=chunk)
    return pl.pallas_call(
        body,
        grid=(b, n // n_rows),
        in_specs=[pl.BlockSpec((1, t, c, n), lambda bi, ji: (bi, 0, 0, 0)),
                  pl.BlockSpec((chunk, chunk), lambda bi, ji: (0, 0))],
        out_specs=pl.BlockSpec((1, n_rows, n), lambda bi, ji: (bi, ji, 0)),
        out_shape=jax.ShapeDtypeStruct((b, n, n), jnp.float32),
        scratch_shapes=[pltpu.VMEM((c, n), jnp.float32),
                        pltpu.VMEM((c, n), jnp.float32)],
    )(xt, sut)
